# baseline (speedup 1.0000x reference)
import functools

import jax
import jax.numpy as jnp
from jax import lax
from jax.experimental import pallas as pl

B, N, D_IN, D_H, D_OUT = 16, 64, 128, 512, 128
EPS = 1e-5
BI = 16


def _edge_kernel(xi_ref, xc_ref, eW1_ref, eb1_ref, eW2_ref, eb2_ref,
                 eg_ref, ebt_ref, eW3_ref, eb3_ref, out_ref):
    xi = xi_ref[0]
    xc = xc_ref[0]
    a = jnp.dot(xi, eW1_ref[:D_IN, :],
                preferred_element_type=jnp.float32) + eb1_ref[0]
    c = jnp.dot(xc, eW1_ref[D_IN:, :], preferred_element_type=jnp.float32)
    h = jnp.maximum(a[:, None, :] + c[None, :, :], 0.0)
    h = h.reshape(BI * N, D_H).astype(jnp.bfloat16)
    h = jnp.dot(h, eW2_ref[...].astype(jnp.bfloat16),
                preferred_element_type=jnp.float32) + eb2_ref[0]
    mu = jnp.mean(h, axis=-1, keepdims=True)
    var = jnp.mean(jnp.square(h - mu), axis=-1, keepdims=True)
    h = (h - mu) * lax.rsqrt(var + EPS) * eg_ref[0] + ebt_ref[0]
    h = jnp.maximum(h, 0.0).astype(jnp.bfloat16)
    ea = jnp.dot(h, eW3_ref[...].astype(jnp.bfloat16),
                 preferred_element_type=jnp.float32)
    ea = ea.reshape(BI, N, D_H)
    i_of_row = pl.program_id(1) * BI + lax.broadcasted_iota(jnp.int32, (BI, N), 0)
    j_idx = lax.broadcasted_iota(jnp.int32, (BI, N), 1)
    keep = (j_idx != i_of_row).astype(jnp.float32)
    agg = jnp.sum(ea * keep[:, :, None], axis=1) + (N - 1) * eb3_ref[0]
    out_ref[0] = agg


def _node_kernel(na_ref, agg_ref, nW1_ref, nb1_ref, nW2_ref, nb2_ref,
                 ng_ref, nbt_ref, nW3_ref, nb3_ref, out_ref):
    h = (jnp.dot(na_ref[...], nW1_ref[:D_IN, :],
                 preferred_element_type=jnp.float32)
         + jnp.dot(agg_ref[...], nW1_ref[D_IN:, :],
                   preferred_element_type=jnp.float32)
         + nb1_ref[0])
    h = jnp.maximum(h, 0.0)
    h = jnp.dot(h, nW2_ref[...], preferred_element_type=jnp.float32) + nb2_ref[0]
    mu = jnp.mean(h, axis=-1, keepdims=True)
    var = jnp.mean(jnp.square(h - mu), axis=-1, keepdims=True)
    h = (h - mu) * lax.rsqrt(var + EPS) * ng_ref[0] + nbt_ref[0]
    h = jnp.maximum(h, 0.0)
    out_ref[...] = jnp.dot(h, nW3_ref[...],
                           preferred_element_type=jnp.float32) + nb3_ref[0]


@jax.jit
def kernel(states, eW1, eb1, eW2, eb2, eg, ebt, eW3, eb3,
           nW1, nb1, nW2, nb2, ng, nbt, nW3, nb3):
    r2 = lambda v: v.reshape(1, -1)
    full = lambda s: pl.BlockSpec(s, lambda b, i: (0,) * len(s))

    agg = pl.pallas_call(
        _edge_kernel,
        grid=(B, N // BI),
        in_specs=[
            pl.BlockSpec((1, BI, D_IN), lambda b, i: (b, i, 0)),
            pl.BlockSpec((1, N, D_IN), lambda b, i: (b, 0, 0)),
            full((2 * D_IN, D_H)),
            full((1, D_H)),
            full((D_H, D_H)),
            full((1, D_H)),
            full((1, D_H)),
            full((1, D_H)),
            full((D_H, D_H)),
            full((1, D_H)),
        ],
        out_specs=pl.BlockSpec((1, BI, D_H), lambda b, i: (b, i, 0)),
        out_shape=jax.ShapeDtypeStruct((B, N, D_H), jnp.float32),
    )(states, states, eW1, r2(eb1), eW2, r2(eb2), r2(eg), r2(ebt),
      eW3, r2(eb3))

    na = states.reshape(B * N, D_IN)
    out = pl.pallas_call(
        _node_kernel,
        out_shape=jax.ShapeDtypeStruct((B * N, D_OUT), jnp.float32),
    )(na, agg.reshape(B * N, D_H), nW1, r2(nb1), nW2, r2(nb2),
      r2(ng), r2(nbt), nW3, r2(nb3))
    return out.reshape(B, N, D_OUT)


# speedup vs baseline: 1.1821x; 1.1821x over previous
import functools

import jax
import jax.numpy as jnp
from jax import lax
from jax.experimental import pallas as pl

B, N, D_IN, D_H, D_OUT = 16, 64, 128, 512, 128
EPS = 1e-5
BI = 32


def _edge_kernel(xi_ref, xc_ref, eW1_ref, eb1_ref, eW2_ref, eb2_ref,
                 eg_ref, ebt_ref, eW3_ref, eb3_ref, out_ref):
    xi = xi_ref[0]
    xc = xc_ref[0]
    a = jnp.dot(xi, eW1_ref[:D_IN, :],
                preferred_element_type=jnp.float32) + eb1_ref[0]
    c = jnp.dot(xc, eW1_ref[D_IN:, :], preferred_element_type=jnp.float32)
    h = jnp.maximum(a[:, None, :] + c[None, :, :], 0.0)
    h = h.reshape(BI * N, D_H)
    h = jnp.dot(h, eW2_ref[...], preferred_element_type=jnp.float32) + eb2_ref[0]
    mu = jnp.mean(h, axis=-1, keepdims=True)
    var = jnp.mean(jnp.square(h - mu), axis=-1, keepdims=True)
    h = (h - mu) * lax.rsqrt(var + EPS) * eg_ref[0] + ebt_ref[0]
    h = jnp.maximum(h, 0.0)
    ea = jnp.dot(h, eW3_ref[...], preferred_element_type=jnp.float32)
    ea = ea.reshape(BI, N, D_H)
    i_of_row = pl.program_id(1) * BI + lax.broadcasted_iota(jnp.int32, (BI, N), 0)
    j_idx = lax.broadcasted_iota(jnp.int32, (BI, N), 1)
    keep = (j_idx != i_of_row).astype(jnp.float32)
    agg = jnp.sum(ea * keep[:, :, None], axis=1) + (N - 1) * eb3_ref[0]
    out_ref[0] = agg


def _node_kernel(na_ref, agg_ref, nW1_ref, nb1_ref, nW2_ref, nb2_ref,
                 ng_ref, nbt_ref, nW3_ref, nb3_ref, out_ref):
    h = (jnp.dot(na_ref[...], nW1_ref[:D_IN, :],
                 preferred_element_type=jnp.float32)
         + jnp.dot(agg_ref[...], nW1_ref[D_IN:, :],
                   preferred_element_type=jnp.float32)
         + nb1_ref[0])
    h = jnp.maximum(h, 0.0)
    h = jnp.dot(h, nW2_ref[...], preferred_element_type=jnp.float32) + nb2_ref[0]
    mu = jnp.mean(h, axis=-1, keepdims=True)
    var = jnp.mean(jnp.square(h - mu), axis=-1, keepdims=True)
    h = (h - mu) * lax.rsqrt(var + EPS) * ng_ref[0] + nbt_ref[0]
    h = jnp.maximum(h, 0.0)
    out_ref[...] = jnp.dot(h, nW3_ref[...],
                           preferred_element_type=jnp.float32) + nb3_ref[0]


@jax.jit
def kernel(states, eW1, eb1, eW2, eb2, eg, ebt, eW3, eb3,
           nW1, nb1, nW2, nb2, ng, nbt, nW3, nb3):
    r2 = lambda v: v.reshape(1, -1)
    full = lambda s: pl.BlockSpec(s, lambda b, i: (0,) * len(s))

    agg = pl.pallas_call(
        _edge_kernel,
        grid=(B, N // BI),
        in_specs=[
            pl.BlockSpec((1, BI, D_IN), lambda b, i: (b, i, 0)),
            pl.BlockSpec((1, N, D_IN), lambda b, i: (b, 0, 0)),
            full((2 * D_IN, D_H)),
            full((1, D_H)),
            full((D_H, D_H)),
            full((1, D_H)),
            full((1, D_H)),
            full((1, D_H)),
            full((D_H, D_H)),
            full((1, D_H)),
        ],
        out_specs=pl.BlockSpec((1, BI, D_H), lambda b, i: (b, i, 0)),
        out_shape=jax.ShapeDtypeStruct((B, N, D_H), jnp.float32),
    )(states, states, eW1, r2(eb1), eW2, r2(eb2), r2(eg), r2(ebt),
      eW3, r2(eb3))

    na = states.reshape(B * N, D_IN)
    out = pl.pallas_call(
        _node_kernel,
        out_shape=jax.ShapeDtypeStruct((B * N, D_OUT), jnp.float32),
    )(na, agg.reshape(B * N, D_H), nW1, r2(nb1), nW2, r2(nb2),
      r2(ng), r2(nbt), nW3, r2(nb3))
    return out.reshape(B, N, D_OUT)


# speedup vs baseline: 1.2143x; 1.0272x over previous
import functools

import jax
import jax.numpy as jnp
from jax import lax
from jax.experimental import pallas as pl

B, N, D_IN, D_H, D_OUT = 16, 64, 128, 512, 128
EPS = 1e-5
BI = 64


def _edge_kernel(xi_ref, xc_ref, eW1_ref, eb1_ref, eW2_ref, eb2_ref,
                 eg_ref, ebt_ref, eW3_ref, eb3_ref, out_ref):
    xi = xi_ref[0]
    xc = xc_ref[0]
    a = jnp.dot(xi, eW1_ref[:D_IN, :],
                preferred_element_type=jnp.float32) + eb1_ref[0]
    c = jnp.dot(xc, eW1_ref[D_IN:, :], preferred_element_type=jnp.float32)
    h = jnp.maximum(a[:, None, :] + c[None, :, :], 0.0)
    h = h.reshape(BI * N, D_H)
    h = jnp.dot(h, eW2_ref[...], preferred_element_type=jnp.float32) + eb2_ref[0]
    mu = jnp.mean(h, axis=-1, keepdims=True)
    var = jnp.mean(jnp.square(h - mu), axis=-1, keepdims=True)
    h = (h - mu) * lax.rsqrt(var + EPS) * eg_ref[0] + ebt_ref[0]
    h = jnp.maximum(h, 0.0)
    ea = jnp.dot(h, eW3_ref[...], preferred_element_type=jnp.float32)
    ea = ea.reshape(BI, N, D_H)
    i_of_row = pl.program_id(1) * BI + lax.broadcasted_iota(jnp.int32, (BI, N), 0)
    j_idx = lax.broadcasted_iota(jnp.int32, (BI, N), 1)
    keep = (j_idx != i_of_row).astype(jnp.float32)
    agg = jnp.sum(ea * keep[:, :, None], axis=1) + (N - 1) * eb3_ref[0]
    out_ref[0] = agg


def _node_kernel(na_ref, agg_ref, nW1_ref, nb1_ref, nW2_ref, nb2_ref,
                 ng_ref, nbt_ref, nW3_ref, nb3_ref, out_ref):
    h = (jnp.dot(na_ref[...], nW1_ref[:D_IN, :],
                 preferred_element_type=jnp.float32)
         + jnp.dot(agg_ref[...], nW1_ref[D_IN:, :],
                   preferred_element_type=jnp.float32)
         + nb1_ref[0])
    h = jnp.maximum(h, 0.0)
    h = jnp.dot(h, nW2_ref[...], preferred_element_type=jnp.float32) + nb2_ref[0]
    mu = jnp.mean(h, axis=-1, keepdims=True)
    var = jnp.mean(jnp.square(h - mu), axis=-1, keepdims=True)
    h = (h - mu) * lax.rsqrt(var + EPS) * ng_ref[0] + nbt_ref[0]
    h = jnp.maximum(h, 0.0)
    out_ref[...] = jnp.dot(h, nW3_ref[...],
                           preferred_element_type=jnp.float32) + nb3_ref[0]


@jax.jit
def kernel(states, eW1, eb1, eW2, eb2, eg, ebt, eW3, eb3,
           nW1, nb1, nW2, nb2, ng, nbt, nW3, nb3):
    r2 = lambda v: v.reshape(1, -1)
    full = lambda s: pl.BlockSpec(s, lambda b, i: (0,) * len(s))

    agg = pl.pallas_call(
        _edge_kernel,
        grid=(B, N // BI),
        in_specs=[
            pl.BlockSpec((1, BI, D_IN), lambda b, i: (b, i, 0)),
            pl.BlockSpec((1, N, D_IN), lambda b, i: (b, 0, 0)),
            full((2 * D_IN, D_H)),
            full((1, D_H)),
            full((D_H, D_H)),
            full((1, D_H)),
            full((1, D_H)),
            full((1, D_H)),
            full((D_H, D_H)),
            full((1, D_H)),
        ],
        out_specs=pl.BlockSpec((1, BI, D_H), lambda b, i: (b, i, 0)),
        out_shape=jax.ShapeDtypeStruct((B, N, D_H), jnp.float32),
    )(states, states, eW1, r2(eb1), eW2, r2(eb2), r2(eg), r2(ebt),
      eW3, r2(eb3))

    na = states.reshape(B * N, D_IN)
    out = pl.pallas_call(
        _node_kernel,
        out_shape=jax.ShapeDtypeStruct((B * N, D_OUT), jnp.float32),
    )(na, agg.reshape(B * N, D_H), nW1, r2(nb1), nW2, r2(nb2),
      r2(ng), r2(nbt), nW3, r2(nb3))
    return out.reshape(B, N, D_OUT)
